# baseline (speedup 1.0000x reference)
import numpy as np

import jax
import jax.numpy as jnp
from jax.experimental import pallas as pl

_N = 8192
_D = 768
_MT = 16
_MY = 32
_BLK = 2048
_C = _MT * _MY

_FOLD_NP = np.zeros((_C, 2 * _MY), np.float32)
for _l in range(_C):
    _FOLD_NP[_l, _l % _MY] = 1.0
_FOLD_NP[:, _MY:] = 1.0


def _half(x, tt, w, bias, fold):
    lane_task = jax.lax.broadcasted_iota(jnp.int32, (1, _C), 1) // _MY
    logits = jax.lax.dot_general(
        x, w, (((1,), (1,)), ((), ())), preferred_element_type=jnp.float32
    ) + bias
    pe = jnp.where(lane_task == tt, jnp.exp(logits), 0.0)
    y = jax.lax.dot_general(
        pe, fold, (((1,), (0,)), ((), ())), preferred_element_type=jnp.float32
    )
    return y[:, :_MY] / y[:, _MY:]


def _body(xa_ref, xb_ref, t_ref, w_ref, b_ref, f_ref, o_ref):
    w = w_ref[...]
    bias = b_ref[...]
    fold = f_ref[...]
    tt = t_ref[...]
    o_ref[0] = _half(xa_ref[...], tt[0], w, bias, fold)
    o_ref[1] = _half(xb_ref[...], tt[1], w, bias, fold)


def kernel(x, t, W, b):
    n, d = x.shape
    w2 = W.reshape(_C, d)
    b2 = b.reshape(1, _C)
    half = n // 2
    t2 = t.reshape(2, half, 1)
    fold = jnp.asarray(_FOLD_NP)
    gh = half // _BLK
    out = pl.pallas_call(
        _body,
        grid=(gh,),
        in_specs=[
            pl.BlockSpec((_BLK, d), lambda i: (i, 0)),
            pl.BlockSpec((_BLK, d), lambda i, _gh=gh: (i + _gh, 0)),
            pl.BlockSpec((2, _BLK, 1), lambda i: (0, i, 0)),
            pl.BlockSpec((_C, d), lambda i: (0, 0)),
            pl.BlockSpec((1, _C), lambda i: (0, 0)),
            pl.BlockSpec((_C, 2 * _MY), lambda i: (0, 0)),
        ],
        out_specs=pl.BlockSpec((2, _BLK, _MY), lambda i: (0, i, 0)),
        out_shape=jax.ShapeDtypeStruct((2, half, _MY), x.dtype),
    )(x, x, t2, w2, b2, fold)
    return out.reshape(n, _MY)


# speedup vs baseline: 1.1319x; 1.1319x over previous
import numpy as np

import jax
import jax.numpy as jnp
from jax.experimental import pallas as pl

_N = 8192
_D = 768
_MT = 16
_MY = 32
_BLK = 2048
_C = _MT * _MY

_FOLD_NP = np.zeros((_C, 2 * _MY), np.float32)
for _l in range(_C):
    _FOLD_NP[_l, _l % _MY] = 1.0
_FOLD_NP[:, _MY:] = 1.0


def _body(x_ref, t_ref, w_ref, b_ref, f_ref, o_ref):
    tt = t_ref[...]
    lane_task = jax.lax.broadcasted_iota(jnp.int32, (1, _C), 1) // _MY
    logits = jax.lax.dot_general(
        x_ref[...], w_ref[...], (((1,), (1,)), ((), ())),
        preferred_element_type=jnp.float32,
    ).astype(jnp.bfloat16)
    z = logits.astype(jnp.float32) + b_ref[...]
    pe = jnp.where(lane_task == tt, jnp.exp(z), 0.0).astype(jnp.bfloat16)
    y = jax.lax.dot_general(
        pe, f_ref[...], (((1,), (0,)), ((), ())), preferred_element_type=jnp.float32
    )
    o_ref[...] = y[:, :_MY] / y[:, _MY:]


def kernel(x, t, W, b):
    n, d = x.shape
    w2 = W.reshape(_C, d)
    b2 = b.reshape(1, _C)
    t2 = t.reshape(n, 1)
    fold = jnp.asarray(_FOLD_NP)
    grid = (n // _BLK,)
    return pl.pallas_call(
        _body,
        grid=grid,
        in_specs=[
            pl.BlockSpec((_BLK, d), lambda i: (i, 0)),
            pl.BlockSpec((_BLK, 1), lambda i: (i, 0)),
            pl.BlockSpec((_C, d), lambda i: (0, 0)),
            pl.BlockSpec((1, _C), lambda i: (0, 0)),
            pl.BlockSpec((_C, 2 * _MY), lambda i: (0, 0)),
        ],
        out_specs=pl.BlockSpec((_BLK, _MY), lambda i: (i, 0)),
        out_shape=jax.ShapeDtypeStruct((n, _MY), x.dtype),
    )(x, t2, w2, b2, fold)
